# baseline (speedup 1.0000x reference)
import functools
import jax
import jax.numpy as jnp
from jax import lax
from jax.experimental import pallas as pl
from jax.experimental.pallas import tpu as pltpu
from jax.experimental.pallas import tpu_sc as plsc

N_BLK = 1024
HID = 32
E_CHUNK = 4000
N_WORKERS = 32


def _prep_tconv(p):
    w = jnp.concatenate([p['wp'], p['wq'], p['wc']], axis=0)
    wt = jnp.transpose(w[:, :, 0, :], (2, 0, 1))
    b = jnp.concatenate([p['bp'], p['bq'], p['bc']])[:, None]
    return wt, b


def _gate(acc, h):
    P = acc[0:h]
    Q = acc[h:2 * h]
    R = acc[2 * h:3 * h]
    return jax.nn.relu(P * jax.nn.sigmoid(Q) + R)


def _k1_body(t_in, h_ref, wt_ref, b_ref, w0_ref, w1_ref, cb_ref, a_ref, z_ref):
    for t in range(t_in - 2):
        acc = b_ref[...]
        for k in range(3):
            acc = acc + jnp.dot(wt_ref[k], h_ref[t + k],
                                preferred_element_type=jnp.float32)
        g = _gate(acc, HID)
        a_ref[t] = jnp.dot(w0_ref[...], g,
                           preferred_element_type=jnp.float32) + cb_ref[...]
        z_ref[t] = jnp.dot(w1_ref[...], g, preferred_element_type=jnp.float32)


def _k1(h, p):
    t_in, c_in, n = h.shape
    wt, b = _prep_tconv(p['tc1'])
    w0t = p['cheb_w'][0].T
    w1t = p['cheb_w'][1].T
    cb = p['cheb_b'][:, None]
    grid = (n // N_BLK,)
    out_sh = jax.ShapeDtypeStruct((t_in - 2, HID, n), jnp.float32)
    full = lambda *s: pl.BlockSpec(s, lambda i: (0,) * len(s))
    return pl.pallas_call(
        functools.partial(_k1_body, t_in),
        grid=grid,
        in_specs=[
            pl.BlockSpec((t_in, c_in, N_BLK), lambda i: (0, 0, i)),
            full(3, 3 * HID, c_in),
            full(3 * HID, 1),
            full(HID, HID),
            full(HID, HID),
            full(HID, 1),
        ],
        out_specs=[
            pl.BlockSpec((t_in - 2, HID, N_BLK), lambda i: (0, 0, i)),
            pl.BlockSpec((t_in - 2, HID, N_BLK), lambda i: (0, 0, i)),
        ],
        out_shape=[out_sh, out_sh],
    )(h, wt, b, w0t, w1t, cb)


def _k2_body(t_in, c_out, final, a_ref, g_ref, wt_ref, b_ref, bng_ref,
             bnb_ref, lw_ref, lb_ref, o_ref):
    t1 = [jax.nn.relu(a_ref[i] + g_ref[i]) for i in range(t_in)]
    t_out = t_in - 2
    hs = []
    for t in range(t_out):
        acc = b_ref[...]
        for k in range(3):
            acc = acc + jnp.dot(wt_ref[k], t1[t + k],
                                preferred_element_type=jnp.float32)
        hs.append(_gate(acc, c_out))
    cnt = t_out * c_out
    mean = sum(jnp.sum(h, axis=0, keepdims=True) for h in hs) / cnt
    var = sum(jnp.sum((h - mean) ** 2, axis=0, keepdims=True) for h in hs) / cnt
    inv = jax.lax.rsqrt(var + 1e-5) * bng_ref[...]
    if not final:
        for t in range(t_out):
            o_ref[t] = (hs[t] - mean) * inv + bnb_ref[...]
    else:
        zs = []
        for t in range(t_out):
            hn = (hs[t] - mean) * inv + bnb_ref[...]
            zs.append(jnp.dot(lw_ref[...], hn,
                              preferred_element_type=jnp.float32) + lb_ref[...])
        m = zs[0]
        for t in range(1, t_out):
            m = jnp.maximum(m, zs[t])
        lse = jnp.log(sum(jnp.exp(z - m) for z in zs)) + m
        for t in range(t_out):
            o_ref[t] = zs[t] - lse


def _k2(a, g, p, lin_w, lin_b, final):
    t_in, _, n = a.shape
    wt, b = _prep_tconv(p['tc2'])
    c_out = wt.shape[1] // 3
    bng = jnp.pad(p['bn_g'], (0, n - p['bn_g'].shape[0]))[None, :]
    bnb = jnp.pad(p['bn_b'], (0, n - p['bn_b'].shape[0]))[None, :]
    oc = 2 if final else c_out
    grid = (n // N_BLK,)
    full = lambda *s: pl.BlockSpec(s, lambda i: (0,) * len(s))
    return pl.pallas_call(
        functools.partial(_k2_body, t_in, c_out, final),
        grid=grid,
        in_specs=[
            pl.BlockSpec((t_in, HID, N_BLK), lambda i: (0, 0, i)),
            pl.BlockSpec((t_in, HID, N_BLK), lambda i: (0, 0, i)),
            full(3, 3 * c_out, HID),
            full(3 * c_out, 1),
            pl.BlockSpec((1, N_BLK), lambda i: (0, i)),
            pl.BlockSpec((1, N_BLK), lambda i: (0, i)),
            full(2, lin_w.shape[1]),
            full(2, 1),
        ],
        out_specs=pl.BlockSpec((t_in - 2, oc, N_BLK), lambda i: (0, 0, i)),
        out_shape=jax.ShapeDtypeStruct((t_in - 2, oc, n), jnp.float32),
    )(a, g, wt, b, bng, bnb, lin_w, lin_b)


def _make_sc_lhat(r_rows, np_, e, interpret=False):
    n_items = r_rows // 4
    n_rounds = (n_items + N_WORKERS - 1) // N_WORKERS
    n_chunks = e // E_CHUNK
    blk = 4 * np_
    mesh = plsc.VectorSubcoreMesh(core_axis_name="c", subcore_axis_name="s",
                                  num_cores=2, num_subcores=16)

    @functools.partial(
        pl.kernel,
        out_type=jax.ShapeDtypeStruct((r_rows * np_,), jnp.float32),
        mesh=mesh,
        interpret=interpret,
        compiler_params=pltpu.CompilerParams(needs_layout_passes=False),
        scratch_types=[
            pltpu.VMEM((blk,), jnp.float32),
            pltpu.VMEM((blk,), jnp.float32),
            pltpu.VMEM((E_CHUNK,), jnp.int32),
            pltpu.VMEM((E_CHUNK,), jnp.int32),
            pltpu.VMEM((E_CHUNK,), jnp.int32),
            pltpu.VMEM((E_CHUNK,), jnp.int32),
            pltpu.VMEM((E_CHUNK,), jnp.float32),
            pltpu.VMEM((E_CHUNK,), jnp.float32),
            pltpu.SemaphoreType.DMA,
            pltpu.SemaphoreType.DMA,
        ],
    )
    def lhat(z_hbm, src_hbm, dst_hbm, nrm_hbm, g_hbm, z_v, o_v, s_v0, s_v1,
             d_v0, d_v1, w_v0, w_v1, sem0, sem1):
        wid = lax.axis_index("s") * 2 + lax.axis_index("c")
        sems = (sem0, sem1)
        s_bufs = (s_v0, s_v1)
        d_bufs = (d_v0, d_v1)
        w_bufs = (w_v0, w_v1)

        def issue(j, b):
            cb = j * E_CHUNK
            pltpu.async_copy(src_hbm.at[pl.ds(cb, E_CHUNK)], s_bufs[b], sems[b])
            pltpu.async_copy(dst_hbm.at[pl.ds(cb, E_CHUNK)], d_bufs[b], sems[b])
            pltpu.async_copy(nrm_hbm.at[pl.ds(cb, E_CHUNK)], w_bufs[b], sems[b])

        def wait3(b):
            pltpu.make_async_copy(src_hbm.at[pl.ds(0, E_CHUNK)], s_bufs[b],
                                  sems[b]).wait()
            pltpu.make_async_copy(dst_hbm.at[pl.ds(0, E_CHUNK)], d_bufs[b],
                                  sems[b]).wait()
            pltpu.make_async_copy(nrm_hbm.at[pl.ds(0, E_CHUNK)], w_bufs[b],
                                  sems[b]).wait()

        for r in range(n_rounds):
            it = wid + r * N_WORKERS

            @pl.when(it < n_items)
            def _():
                base = it * blk
                pltpu.sync_copy(z_hbm.at[pl.ds(base, blk)], z_v)

                @plsc.parallel_loop(0, blk // 16, unroll=8)
                def _zero(i):
                    o_v[pl.ds(i * 16, 16)] = jnp.zeros((16,), jnp.float32)

                issue(0, 0)
                issue(1, 1)

                def chunk_pair(jo, _):
                    for b in range(2):
                        j = jo * 2 + b
                        wait3(b)

                        @plsc.parallel_loop(0, E_CHUNK // 16, unroll=8)
                        def _grp(g):
                            o = g * 16
                            s16 = s_bufs[b][pl.ds(o, 16)]
                            d16 = d_bufs[b][pl.ds(o, 16)]
                            w16 = w_bufs[b][pl.ds(o, 16)]
                            for c in range(4):
                                vals = plsc.load_gather(z_v, [s16 + c * np_])
                                plsc.addupdate_scatter(o_v, [d16 + c * np_],
                                                       vals * w16)

                        @pl.when(j + 2 < n_chunks)
                        def _next():
                            issue(j + 2, b)

                    return 0

                lax.fori_loop(0, n_chunks // 2, chunk_pair, 0)
                pltpu.sync_copy(o_v, g_hbm.at[pl.ds(base, blk)])

    return lhat


def _make_sc_norm(np_, e, interpret=False):
    per_sc = e // 16
    per_w = e // 32
    dc = 2000
    mesh = plsc.VectorSubcoreMesh(core_axis_name="c", subcore_axis_name="s",
                                  num_cores=2, num_subcores=16)

    rows = np_ // 128

    @functools.partial(
        pl.kernel,
        out_type=jax.ShapeDtypeStruct((e,), jnp.float32),
        mesh=mesh,
        interpret=interpret,
        compiler_params=pltpu.CompilerParams(needs_layout_passes=False),
        scratch_types=[
            pltpu.VMEM((rows, 128), jnp.float32),
            pltpu.VMEM((rows,), jnp.int32),
            pltpu.VMEM((dc,), jnp.int32),
            pltpu.VMEM((dc,), jnp.int32),
            pltpu.VMEM((dc,), jnp.float32),
            pltpu.VMEM((dc,), jnp.float32),
            pltpu.VMEM_SHARED((rows, 128), jnp.float32),
        ],
    )
    def prep(src_hbm, dst_hbm, w_hbm, nrm_hbm, dl, ix_v, s_v, d_v, w_v, o_v,
             deg_sh):
        s = lax.axis_index("s")
        wid = s * 2 + lax.axis_index("c")

        for j in range(rows // 16):
            ix_v[pl.ds(j * 16, 16)] = lax.iota(jnp.int32, 16) + j * 16

        def _zero_rows(lo, hi):
            @plsc.parallel_loop(lo, hi, unroll=8)
            def _z(i):
                r = lax.shift_right_logical(i, 3)
                col = (i & 7) * 16
                dl[r, pl.ds(col, 16)] = jnp.zeros((16,), jnp.float32)

        _zero_rows(0, rows * 8)

        @pl.when(s == 0)
        def _():
            pltpu.sync_copy(dl, deg_sh)

        plsc.subcore_barrier()

        def chunk_a(ck, _):
            cb = s * per_sc + ck * dc
            pltpu.sync_copy(src_hbm.at[pl.ds(cb, dc)], s_v)
            pltpu.sync_copy(w_hbm.at[pl.ds(cb, dc)], w_v)

            @plsc.parallel_loop(0, dc // 16, unroll=4)
            def _g(g):
                o = g * 16
                s16 = s_v[pl.ds(o, 16)]
                plsc.addupdate_scatter(
                    dl, [lax.shift_right_logical(s16, 7), s16 & 127],
                    w_v[pl.ds(o, 16)])

            return 0

        lax.fori_loop(0, per_sc // dc, chunk_a, 0)

        pltpu.sync_copy(dl, deg_sh.at[ix_v], add=True)
        plsc.subcore_barrier()
        pltpu.sync_copy(deg_sh, dl)

        @plsc.parallel_loop(0, rows * 8, unroll=4)
        def _rsqrt(i):
            r = lax.shift_right_logical(i, 3)
            col = (i & 7) * 16
            d = dl[r, pl.ds(col, 16)]
            bits = jnp.int32(0x5F3759DF) - lax.shift_right_logical(
                plsc.bitcast(d, jnp.int32), 1)
            h = plsc.bitcast(bits, jnp.float32)
            for _ in range(3):
                h = h * (1.5 - 0.5 * d * h * h)
            dl[r, pl.ds(col, 16)] = jnp.where(d > 0, h, 0.0)

        def chunk_b(ck, _):
            cb = wid * per_w + ck * dc
            pltpu.sync_copy(src_hbm.at[pl.ds(cb, dc)], s_v)
            pltpu.sync_copy(dst_hbm.at[pl.ds(cb, dc)], d_v)
            pltpu.sync_copy(w_hbm.at[pl.ds(cb, dc)], w_v)

            @plsc.parallel_loop(0, dc // 16, unroll=4)
            def _g(g):
                o = g * 16
                s16 = s_v[pl.ds(o, 16)]
                d16 = d_v[pl.ds(o, 16)]
                da = plsc.load_gather(
                    dl, [lax.shift_right_logical(s16, 7), s16 & 127])
                db = plsc.load_gather(
                    dl, [lax.shift_right_logical(d16, 7), d16 & 127])
                o_v[pl.ds(o, 16)] = -(w_v[pl.ds(o, 16)] * da * db)

            pltpu.sync_copy(o_v, nrm_hbm.at[pl.ds(cb, dc)])
            return 0

        lax.fori_loop(0, per_w // dc, chunk_b, 0)

    return prep


def _lhat(z, src, dst, norm):
    t, c, n = z.shape
    fn = _make_sc_lhat(t * c, n, src.shape[0])
    g = fn(z.reshape(-1), src, dst, norm)
    return g.reshape(t, c, n)


def kernel(x, edge_index, edge_weight, params):
    src = edge_index[0]
    dst = edge_index[1]
    n = x.shape[2]
    n_pad = (n + N_BLK - 1) // N_BLK * N_BLK
    norm = _make_sc_norm(n_pad, src.shape[0])(src, dst, edge_weight)
    h = jnp.transpose(x[0], (0, 2, 1))
    h = jnp.pad(h, ((0, 0), (0, 0), (0, n_pad - n)))
    for key in ('stc1', 'stc2'):
        p = params[key]
        final = key == 'stc2'
        a, z = _k1(h, p)
        g = _lhat(z, src, dst, norm)
        h = _k2(a, g, p, params['lin_w'], params['lin_b'][:, None], final)
    out = jnp.transpose(h, (0, 2, 1))[None]
    return out[:, :, :n, :]


# speedup vs baseline: 1.0191x; 1.0191x over previous
import functools
import jax
import jax.numpy as jnp
from jax import lax
from jax.experimental import pallas as pl
from jax.experimental.pallas import tpu as pltpu
from jax.experimental.pallas import tpu_sc as plsc

N_BLK = 1024
HID = 32
E_CHUNK = 4000
N_WORKERS = 32


def _prep_tconv(p):
    w = jnp.concatenate([p['wp'], p['wq'], p['wc']], axis=0)
    wt = jnp.transpose(w[:, :, 0, :], (2, 0, 1))
    b = jnp.concatenate([p['bp'], p['bq'], p['bc']])[:, None]
    return wt, b


def _gate(acc, h):
    P = acc[0:h]
    Q = acc[h:2 * h]
    R = acc[2 * h:3 * h]
    return jax.nn.relu(P * jax.nn.sigmoid(Q) + R)


def _k1_body(t_in, h_ref, wt_ref, b_ref, w0_ref, w1_ref, cb_ref, a_ref, z_ref):
    for t in range(t_in - 2):
        acc = b_ref[...]
        for k in range(3):
            acc = acc + jnp.dot(wt_ref[k], h_ref[t + k],
                                preferred_element_type=jnp.float32)
        g = _gate(acc, HID)
        a_ref[t] = jnp.dot(w0_ref[...], g,
                           preferred_element_type=jnp.float32) + cb_ref[...]
        z_ref[t] = jnp.dot(w1_ref[...], g, preferred_element_type=jnp.float32)


def _k1(h, p):
    t_in, c_in, n = h.shape
    wt, b = _prep_tconv(p['tc1'])
    w0t = p['cheb_w'][0].T
    w1t = p['cheb_w'][1].T
    cb = p['cheb_b'][:, None]
    grid = (n // N_BLK,)
    out_sh = jax.ShapeDtypeStruct((t_in - 2, HID, n), jnp.float32)
    full = lambda *s: pl.BlockSpec(s, lambda i: (0,) * len(s))
    return pl.pallas_call(
        functools.partial(_k1_body, t_in),
        grid=grid,
        in_specs=[
            pl.BlockSpec((t_in, c_in, N_BLK), lambda i: (0, 0, i)),
            full(3, 3 * HID, c_in),
            full(3 * HID, 1),
            full(HID, HID),
            full(HID, HID),
            full(HID, 1),
        ],
        out_specs=[
            pl.BlockSpec((t_in - 2, HID, N_BLK), lambda i: (0, 0, i)),
            pl.BlockSpec((t_in - 2, HID, N_BLK), lambda i: (0, 0, i)),
        ],
        out_shape=[out_sh, out_sh],
    )(h, wt, b, w0t, w1t, cb)


def _k2_body(t_in, c_out, final, a_ref, g_ref, wt_ref, b_ref, bng_ref,
             bnb_ref, lw_ref, lb_ref, o_ref):
    t1 = [jax.nn.relu(a_ref[i] + g_ref[i]) for i in range(t_in)]
    t_out = t_in - 2
    hs = []
    for t in range(t_out):
        acc = b_ref[...]
        for k in range(3):
            acc = acc + jnp.dot(wt_ref[k], t1[t + k],
                                preferred_element_type=jnp.float32)
        hs.append(_gate(acc, c_out))
    cnt = t_out * c_out
    mean = sum(jnp.sum(h, axis=0, keepdims=True) for h in hs) / cnt
    var = sum(jnp.sum((h - mean) ** 2, axis=0, keepdims=True) for h in hs) / cnt
    inv = jax.lax.rsqrt(var + 1e-5) * bng_ref[...]
    if not final:
        for t in range(t_out):
            o_ref[t] = (hs[t] - mean) * inv + bnb_ref[...]
    else:
        zs = []
        for t in range(t_out):
            hn = (hs[t] - mean) * inv + bnb_ref[...]
            zs.append(jnp.dot(lw_ref[...], hn,
                              preferred_element_type=jnp.float32) + lb_ref[...])
        m = zs[0]
        for t in range(1, t_out):
            m = jnp.maximum(m, zs[t])
        lse = jnp.log(sum(jnp.exp(z - m) for z in zs)) + m
        for t in range(t_out):
            o_ref[t] = zs[t] - lse


def _k2(a, g, p, lin_w, lin_b, final):
    t_in, _, n = a.shape
    wt, b = _prep_tconv(p['tc2'])
    c_out = wt.shape[1] // 3
    bng = jnp.pad(p['bn_g'], (0, n - p['bn_g'].shape[0]))[None, :]
    bnb = jnp.pad(p['bn_b'], (0, n - p['bn_b'].shape[0]))[None, :]
    oc = 2 if final else c_out
    grid = (n // N_BLK,)
    full = lambda *s: pl.BlockSpec(s, lambda i: (0,) * len(s))
    return pl.pallas_call(
        functools.partial(_k2_body, t_in, c_out, final),
        grid=grid,
        in_specs=[
            pl.BlockSpec((t_in, HID, N_BLK), lambda i: (0, 0, i)),
            pl.BlockSpec((t_in, HID, N_BLK), lambda i: (0, 0, i)),
            full(3, 3 * c_out, HID),
            full(3 * c_out, 1),
            pl.BlockSpec((1, N_BLK), lambda i: (0, i)),
            pl.BlockSpec((1, N_BLK), lambda i: (0, i)),
            full(2, lin_w.shape[1]),
            full(2, 1),
        ],
        out_specs=pl.BlockSpec((t_in - 2, oc, N_BLK), lambda i: (0, 0, i)),
        out_shape=jax.ShapeDtypeStruct((t_in - 2, oc, n), jnp.float32),
    )(a, g, wt, b, bng, bnb, lin_w, lin_b)


def _make_sc_lhat(r_rows, np_, e, interpret=False):
    n_items = r_rows // 4
    n_rounds = (n_items + N_WORKERS - 1) // N_WORKERS
    n_chunks = e // E_CHUNK
    blk = 4 * np_
    mesh = plsc.VectorSubcoreMesh(core_axis_name="c", subcore_axis_name="s",
                                  num_cores=2, num_subcores=16)

    @functools.partial(
        pl.kernel,
        out_type=jax.ShapeDtypeStruct((r_rows * np_,), jnp.float32),
        mesh=mesh,
        interpret=interpret,
        compiler_params=pltpu.CompilerParams(needs_layout_passes=False),
        scratch_types=[
            pltpu.VMEM((blk,), jnp.float32),
            pltpu.VMEM((blk,), jnp.float32),
            pltpu.VMEM((E_CHUNK,), jnp.int32),
            pltpu.VMEM((E_CHUNK,), jnp.int32),
            pltpu.VMEM((E_CHUNK,), jnp.int32),
            pltpu.VMEM((E_CHUNK,), jnp.int32),
            pltpu.VMEM((E_CHUNK,), jnp.float32),
            pltpu.VMEM((E_CHUNK,), jnp.float32),
            pltpu.SemaphoreType.DMA,
            pltpu.SemaphoreType.DMA,
        ],
    )
    def lhat(z_hbm, src_hbm, dst_hbm, nrm_hbm, g_hbm, z_v, o_v, s_v0, s_v1,
             d_v0, d_v1, w_v0, w_v1, sem0, sem1):
        wid = lax.axis_index("s") * 2 + lax.axis_index("c")
        sems = (sem0, sem1)
        s_bufs = (s_v0, s_v1)
        d_bufs = (d_v0, d_v1)
        w_bufs = (w_v0, w_v1)

        def issue(j, b):
            cb = j * E_CHUNK
            pltpu.async_copy(src_hbm.at[pl.ds(cb, E_CHUNK)], s_bufs[b], sems[b])
            pltpu.async_copy(dst_hbm.at[pl.ds(cb, E_CHUNK)], d_bufs[b], sems[b])
            pltpu.async_copy(nrm_hbm.at[pl.ds(cb, E_CHUNK)], w_bufs[b], sems[b])

        def wait3(b):
            pltpu.make_async_copy(src_hbm.at[pl.ds(0, E_CHUNK)], s_bufs[b],
                                  sems[b]).wait()
            pltpu.make_async_copy(dst_hbm.at[pl.ds(0, E_CHUNK)], d_bufs[b],
                                  sems[b]).wait()
            pltpu.make_async_copy(nrm_hbm.at[pl.ds(0, E_CHUNK)], w_bufs[b],
                                  sems[b]).wait()

        for r in range(n_rounds):
            it = wid + r * N_WORKERS

            @pl.when(it < n_items)
            def _():
                base = it * blk
                pltpu.sync_copy(z_hbm.at[pl.ds(base, blk)], z_v)

                @plsc.parallel_loop(0, blk // 16, unroll=8)
                def _zero(i):
                    o_v[pl.ds(i * 16, 16)] = jnp.zeros((16,), jnp.float32)

                issue(0, 0)
                issue(1, 1)

                def chunk_pair(jo, _):
                    for b in range(2):
                        j = jo * 2 + b
                        wait3(b)

                        @plsc.parallel_loop(0, E_CHUNK // 16, unroll=4)
                        def _grp(g):
                            o = g * 16
                            s16 = s_bufs[b][pl.ds(o, 16)]
                            d16 = d_bufs[b][pl.ds(o, 16)]
                            w16 = w_bufs[b][pl.ds(o, 16)]
                            for c in range(4):
                                vals = plsc.load_gather(z_v, [s16 + c * np_])
                                plsc.addupdate_scatter(o_v, [d16 + c * np_],
                                                       vals * w16)

                        @pl.when(j + 2 < n_chunks)
                        def _next():
                            issue(j + 2, b)

                    return 0

                lax.fori_loop(0, n_chunks // 2, chunk_pair, 0)
                pltpu.sync_copy(o_v, g_hbm.at[pl.ds(base, blk)])

    return lhat


def _make_sc_norm(np_, e, interpret=False):
    per_sc = e // 16
    per_w = e // 32
    dc = 2000
    mesh = plsc.VectorSubcoreMesh(core_axis_name="c", subcore_axis_name="s",
                                  num_cores=2, num_subcores=16)

    rows = np_ // 128

    @functools.partial(
        pl.kernel,
        out_type=jax.ShapeDtypeStruct((e,), jnp.float32),
        mesh=mesh,
        interpret=interpret,
        compiler_params=pltpu.CompilerParams(needs_layout_passes=False),
        scratch_types=[
            pltpu.VMEM((rows, 128), jnp.float32),
            pltpu.VMEM((rows,), jnp.int32),
            pltpu.VMEM((dc,), jnp.int32),
            pltpu.VMEM((dc,), jnp.int32),
            pltpu.VMEM((dc,), jnp.float32),
            pltpu.VMEM((dc,), jnp.float32),
            pltpu.VMEM_SHARED((rows, 128), jnp.float32),
        ],
    )
    def prep(src_hbm, dst_hbm, w_hbm, nrm_hbm, dl, ix_v, s_v, d_v, w_v, o_v,
             deg_sh):
        s = lax.axis_index("s")
        wid = s * 2 + lax.axis_index("c")

        for j in range(rows // 16):
            ix_v[pl.ds(j * 16, 16)] = lax.iota(jnp.int32, 16) + j * 16

        def _zero_rows(lo, hi):
            @plsc.parallel_loop(lo, hi, unroll=8)
            def _z(i):
                r = lax.shift_right_logical(i, 3)
                col = (i & 7) * 16
                dl[r, pl.ds(col, 16)] = jnp.zeros((16,), jnp.float32)

        _zero_rows(0, rows * 8)

        @pl.when(s == 0)
        def _():
            pltpu.sync_copy(dl, deg_sh)

        plsc.subcore_barrier()

        def chunk_a(ck, _):
            cb = s * per_sc + ck * dc
            pltpu.sync_copy(src_hbm.at[pl.ds(cb, dc)], s_v)
            pltpu.sync_copy(w_hbm.at[pl.ds(cb, dc)], w_v)

            @plsc.parallel_loop(0, dc // 16, unroll=4)
            def _g(g):
                o = g * 16
                s16 = s_v[pl.ds(o, 16)]
                plsc.addupdate_scatter(
                    dl, [lax.shift_right_logical(s16, 7), s16 & 127],
                    w_v[pl.ds(o, 16)])

            return 0

        lax.fori_loop(0, per_sc // dc, chunk_a, 0)

        pltpu.sync_copy(dl, deg_sh.at[ix_v], add=True)
        plsc.subcore_barrier()
        pltpu.sync_copy(deg_sh, dl)

        @plsc.parallel_loop(0, rows * 8, unroll=4)
        def _rsqrt(i):
            r = lax.shift_right_logical(i, 3)
            col = (i & 7) * 16
            d = dl[r, pl.ds(col, 16)]
            bits = jnp.int32(0x5F3759DF) - lax.shift_right_logical(
                plsc.bitcast(d, jnp.int32), 1)
            h = plsc.bitcast(bits, jnp.float32)
            for _ in range(3):
                h = h * (1.5 - 0.5 * d * h * h)
            dl[r, pl.ds(col, 16)] = jnp.where(d > 0, h, 0.0)

        def chunk_b(ck, _):
            cb = wid * per_w + ck * dc
            pltpu.sync_copy(src_hbm.at[pl.ds(cb, dc)], s_v)
            pltpu.sync_copy(dst_hbm.at[pl.ds(cb, dc)], d_v)
            pltpu.sync_copy(w_hbm.at[pl.ds(cb, dc)], w_v)

            @plsc.parallel_loop(0, dc // 16, unroll=4)
            def _g(g):
                o = g * 16
                s16 = s_v[pl.ds(o, 16)]
                d16 = d_v[pl.ds(o, 16)]
                da = plsc.load_gather(
                    dl, [lax.shift_right_logical(s16, 7), s16 & 127])
                db = plsc.load_gather(
                    dl, [lax.shift_right_logical(d16, 7), d16 & 127])
                o_v[pl.ds(o, 16)] = -(w_v[pl.ds(o, 16)] * da * db)

            pltpu.sync_copy(o_v, nrm_hbm.at[pl.ds(cb, dc)])
            return 0

        lax.fori_loop(0, per_w // dc, chunk_b, 0)

    return prep


def _lhat(z, src, dst, norm):
    t, c, n = z.shape
    fn = _make_sc_lhat(t * c, n, src.shape[0])
    g = fn(z.reshape(-1), src, dst, norm)
    return g.reshape(t, c, n)


def kernel(x, edge_index, edge_weight, params):
    src = edge_index[0]
    dst = edge_index[1]
    n = x.shape[2]
    n_pad = (n + N_BLK - 1) // N_BLK * N_BLK
    norm = _make_sc_norm(n_pad, src.shape[0])(src, dst, edge_weight)
    h = jnp.transpose(x[0], (0, 2, 1))
    h = jnp.pad(h, ((0, 0), (0, 0), (0, n_pad - n)))
    for key in ('stc1', 'stc2'):
        p = params[key]
        final = key == 'stc2'
        a, z = _k1(h, p)
        g = _lhat(z, src, dst, norm)
        h = _k2(a, g, p, params['lin_w'], params['lin_b'][:, None], final)
    out = jnp.transpose(h, (0, 2, 1))[None]
    return out[:, :, :n, :]
